# Initial kernel scaffold; baseline (speedup 1.0000x reference)
import jax
import jax.numpy as jnp
from jax.experimental import pallas as pl


def kernel(x, edge_index, batch, W1, b1, gamma, beta, Wc1, as1, ad1, bc1, Wc2, as2, ad2, bc2, Wg, bg, Wq1, bq1, Wq2, bq2):
    raise NotImplementedError("write your pallas kernel here")


# speedup vs baseline: 17.5078x; 17.5078x over previous
import functools

import jax
import jax.numpy as jnp
from jax import lax
from jax.experimental import pallas as pl
from jax.experimental.pallas import tpu as pltpu
from jax.experimental.pallas import tpu_sc as plsc

N_NODES = 50000
NP = 50176
H = 64
HH = 32
G = 8
A = 10

NC, NS = 2, 16
NW = NC * NS
EP = 851968
EROWS = EP // 128
ROWS_PER_TILE = EROWS // NW
CHUNK_ROWS = 16
N_CHUNKS = ROWS_PER_TILE // CHUNK_ROWS
STRIPE = NP // NS

BR = 3136
NB = NP // BR

_f32 = jnp.float32
_i32 = jnp.int32


def _gat_head(h, Wc_ref, as_ref, ad_ref, g128_ref, s_ref, d_ref,
              smax_ref, dmax_ref, k):
    hw = jnp.dot(h, Wc_ref[...], precision=lax.Precision.HIGHEST)
    g128_ref[...] = jnp.concatenate([hw, jnp.zeros((BR, H), _f32)], axis=1)
    sv = jnp.sum(hw * as_ref[...], axis=1, keepdims=True)
    dv = jnp.sum(hw * ad_ref[...], axis=1, keepdims=True)
    s_ref[...] = sv
    d_ref[...] = dv
    sm = jnp.full((8, 128), jnp.max(sv), _f32)
    dm = jnp.full((8, 128), jnp.max(dv), _f32)

    @pl.when(k == 0)
    def _():
        smax_ref[...] = sm
        dmax_ref[...] = dm

    @pl.when(k > 0)
    def _():
        smax_ref[...] = jnp.maximum(smax_ref[...], sm)
        dmax_ref[...] = jnp.maximum(dmax_ref[...], dm)


def _prep_body(x_ref, W1_ref, b1_ref, gam_ref, bet_ref, Wc_ref, as_ref, ad_ref,
               g128_ref, s_ref, d_ref, smax_ref, dmax_ref):
    k = pl.program_id(0)
    h = jnp.maximum(jnp.dot(x_ref[...], W1_ref[...], precision=lax.Precision.HIGHEST) + b1_ref[...], 0.0)
    mu = jnp.mean(h, axis=1, keepdims=True)
    var = jnp.mean((h - mu) ** 2, axis=1, keepdims=True)
    h = (h - mu) / jnp.sqrt(var + 1e-5) * gam_ref[...] + bet_ref[...]
    _gat_head(h, Wc_ref, as_ref, ad_ref, g128_ref, s_ref, d_ref,
              smax_ref, dmax_ref, k)


def _mid_body(alo_ref, ahi_ref, d0_ref, d1_ref, bc_ref, Wc_ref, as_ref, ad_ref,
              g128_ref, s_ref, d_ref, smax_ref, dmax_ref):
    k = pl.program_id(0)
    agg = jnp.concatenate([alo_ref[...], ahi_ref[...]], axis=1)
    den = d0_ref[...] + d1_ref[...] + 1e-16
    h = jnp.maximum(agg / den + bc_ref[...], 0.0)
    _gat_head(h, Wc_ref, as_ref, ad_ref, g128_ref, s_ref, d_ref,
              smax_ref, dmax_ref, k)


_HEAD_OUT_SHAPES = (
    jax.ShapeDtypeStruct((NP, 128), _f32),
    jax.ShapeDtypeStruct((NP, 1), _f32),
    jax.ShapeDtypeStruct((NP, 1), _f32),
    jax.ShapeDtypeStruct((8, 128), _f32),
    jax.ShapeDtypeStruct((8, 128), _f32),
)

_HEAD_OUT_SPECS = (
    pl.BlockSpec((BR, 128), lambda k: (k, 0)),
    pl.BlockSpec((BR, 1), lambda k: (k, 0)),
    pl.BlockSpec((BR, 1), lambda k: (k, 0)),
    pl.BlockSpec((8, 128), lambda k: (0, 0)),
    pl.BlockSpec((8, 128), lambda k: (0, 0)),
)

_VEC_SPEC = pl.BlockSpec((1, H), lambda k: (0, 0))
_MAT_SPEC = pl.BlockSpec((H, H), lambda k: (0, 0))
_ROWS_SPEC = pl.BlockSpec((BR, HH), lambda k: (k, 0))
_COL_SPEC = pl.BlockSpec((BR, 1), lambda k: (k, 0))


def _prep_call(x8, W1p, b1, gam, bet, Wc, a_s, a_d):
    return pl.pallas_call(
        _prep_body,
        grid=(NB,),
        in_specs=[
            pl.BlockSpec((BR, 8), lambda k: (k, 0)),
            pl.BlockSpec((8, H), lambda k: (0, 0)),
            _VEC_SPEC, _VEC_SPEC, _VEC_SPEC,
            _MAT_SPEC, _VEC_SPEC, _VEC_SPEC,
        ],
        out_specs=list(_HEAD_OUT_SPECS),
        out_shape=list(_HEAD_OUT_SHAPES),
    )(x8, W1p, b1, gam, bet, Wc, a_s, a_d)


def _mid_call(alo, ahi, d0, d1, bc, Wc, a_s, a_d):
    return pl.pallas_call(
        _mid_body,
        grid=(NB,),
        in_specs=[
            _ROWS_SPEC, _ROWS_SPEC, _COL_SPEC, _COL_SPEC,
            _VEC_SPEC, _MAT_SPEC, _VEC_SPEC, _VEC_SPEC,
        ],
        out_specs=list(_HEAD_OUT_SPECS),
        out_shape=list(_HEAD_OUT_SHAPES),
    )(alo, ahi, d0, d1, bc, Wc, a_s, a_d)


def _poolA_body(alo_ref, ahi_ref, d0_ref, d1_ref, bc_ref, wgt_ref, bg_ref,
                batch_ref, h2_ref, gate_ref, gmax_ref):
    k = pl.program_id(0)
    agg = jnp.concatenate([alo_ref[...], ahi_ref[...]], axis=1)
    den = d0_ref[...] + d1_ref[...] + 1e-16
    h2 = jnp.maximum(agg / den + bc_ref[...], 0.0)
    h2_ref[...] = h2
    gate = jnp.sum(h2 * wgt_ref[...], axis=1, keepdims=True) + bg_ref[...]
    gate_ref[...] = gate
    b = batch_ref[...]
    parts = []
    for g in range(G):
        mg = jnp.max(jnp.where(b == float(g), gate, -3.4e38))
        parts.append(jnp.full((1, 128), mg, _f32))
    gm = jnp.concatenate(parts, axis=0)

    @pl.when(k == 0)
    def _():
        gmax_ref[...] = gm

    @pl.when(k > 0)
    def _():
        gmax_ref[...] = jnp.maximum(gmax_ref[...], gm)


def _poolA_call(alo, ahi, d0, d1, bc, wgt, bg, batchf):
    return pl.pallas_call(
        _poolA_body,
        grid=(NB,),
        in_specs=[
            _ROWS_SPEC, _ROWS_SPEC, _COL_SPEC, _COL_SPEC,
            _VEC_SPEC, _VEC_SPEC, pl.BlockSpec((1, 1), lambda k: (0, 0)),
            _COL_SPEC,
        ],
        out_specs=[
            pl.BlockSpec((BR, H), lambda k: (k, 0)),
            _COL_SPEC,
            pl.BlockSpec((8, 128), lambda k: (0, 0)),
        ],
        out_shape=[
            jax.ShapeDtypeStruct((NP, H), _f32),
            jax.ShapeDtypeStruct((NP, 1), _f32),
            jax.ShapeDtypeStruct((8, 128), _f32),
        ],
    )(alo, ahi, d0, d1, bc, wgt, bg, batchf)


def _poolB_body(h2_ref, gate_ref, gmax_ref, batch_ref, Wq1_ref, bq1_ref,
                Wq2_ref, bq2_ref, q_ref, pnum_ref, gden_ref):
    k = pl.program_id(0)

    @pl.when(k == 0)
    def _():
        pnum_ref[...] = jnp.zeros((G, H), _f32)
        gden_ref[...] = jnp.zeros((G, 128), _f32)

    b = batch_ref[...]
    gate = gate_ref[...]
    gm = gmax_ref[...]
    rm = jnp.zeros_like(gate)
    for g in range(G):
        rm = rm + jnp.where(b == float(g), gm[g:g + 1, 0:1], 0.0)
    ge = jnp.exp(jnp.minimum(gate - rm, 60.0))
    h2 = h2_ref[...]
    prows, drows = [], []
    for g in range(G):
        geg = jnp.where(b == float(g), ge, 0.0)
        drows.append(jnp.full((1, 128), jnp.sum(geg), _f32))
        prows.append(jnp.sum(geg * h2, axis=0, keepdims=True))
    pnum_ref[...] += jnp.concatenate(prows, axis=0)
    gden_ref[...] += jnp.concatenate(drows, axis=0)

    @pl.when(k == NB - 1)
    def _():
        pooled = pnum_ref[...] / (gden_ref[...][:, 0:1] + 1e-16)
        qh = jnp.maximum(jnp.dot(pooled, Wq1_ref[...], precision=lax.Precision.HIGHEST) + bq1_ref[...], 0.0)
        q_ref[...] = jnp.dot(qh, Wq2_ref[...], precision=lax.Precision.HIGHEST) + bq2_ref[...]


def _poolB_call(h2, gate, gmax, batchf, Wq1, bq1, Wq2, bq2):
    return pl.pallas_call(
        _poolB_body,
        grid=(NB,),
        in_specs=[
            pl.BlockSpec((BR, H), lambda k: (k, 0)),
            _COL_SPEC,
            pl.BlockSpec((8, 128), lambda k: (0, 0)),
            _COL_SPEC,
            _MAT_SPEC, _VEC_SPEC,
            pl.BlockSpec((H, A), lambda k: (0, 0)),
            pl.BlockSpec((1, A), lambda k: (0, 0)),
        ],
        out_specs=pl.BlockSpec((G, A), lambda k: (0, 0)),
        out_shape=jax.ShapeDtypeStruct((G, A), _f32),
        scratch_shapes=[
            pltpu.VMEM((G, H), _f32),
            pltpu.VMEM((G, 128), _f32),
        ],
    )(h2, gate, gmax, batchf, Wq1, bq1, Wq2, bq2)


@functools.lru_cache(maxsize=None)
def _sc_mesh():
    return plsc.VectorSubcoreMesh(
        core_axis_name="c", subcore_axis_name="s",
        num_cores=NC, num_subcores=NS)


def _edge_scalar_body(s_hbm, d_hbm, src_hbm, dst_hbm, m_hbm,
                      ee_hbm, den_hbm,
                      s_v, d_v, srcb, dstb, eeb, zer_v, m_v, den_sh):
    c = lax.axis_index("c")
    t = lax.axis_index("s")
    wid = c * NS + t

    def zfill(i, carry):
        zer_v[pl.ds(i * 16, 16)] = jnp.zeros((16,), _f32)
        return carry

    lax.fori_loop(0, STRIPE // 16, zfill, 0)
    pltpu.sync_copy(zer_v, den_sh.at[pl.ds(t * STRIPE, STRIPE)])
    pltpu.sync_copy(s_hbm, s_v)
    pltpu.sync_copy(d_hbm, d_v)
    pltpu.sync_copy(m_hbm, m_v)
    plsc.subcore_barrier()
    m = m_v[...]
    row0 = wid * ROWS_PER_TILE

    def chunk(ci, carry):
        r = row0 + ci * CHUNK_ROWS
        pltpu.sync_copy(src_hbm.at[pl.ds(r, CHUNK_ROWS)], srcb)
        pltpu.sync_copy(dst_hbm.at[pl.ds(r, CHUNK_ROWS)], dstb)

        def grp(tt, carry2):
            j = tt // 8
            i = (tt % 8) * 16
            si = srcb[j, pl.ds(i, 16)]
            di = dstb[j, pl.ds(i, 16)]
            sv = plsc.load_gather(s_v, [si])
            dv = plsc.load_gather(d_v, [di])
            e = sv + dv
            e = jnp.where(e > 0, e, e * 0.2) - m
            eeb[j, pl.ds(i, 16)] = jnp.exp(e)
            return carry2

        lax.fori_loop(0, CHUNK_ROWS * 8, grp, 0)
        pltpu.sync_copy(eeb, ee_hbm.at[pl.ds(r, CHUNK_ROWS)])

        def srow(j, carry2):
            pltpu.sync_copy(eeb.at[j], den_sh.at[dstb.at[j]], add=True)
            return carry2

        lax.fori_loop(0, CHUNK_ROWS, srow, 0)
        return carry

    lax.fori_loop(0, N_CHUNKS, chunk, 0)
    plsc.subcore_barrier()
    pltpu.sync_copy(den_sh.at[pl.ds(t * STRIPE, STRIPE)], zer_v)
    pltpu.sync_copy(zer_v, den_hbm.at[pl.ds(c * NP + t * STRIPE, STRIPE)])


@functools.lru_cache(maxsize=None)
def _edge_scalar_kernel():
    return pl.kernel(
        _edge_scalar_body,
        out_type=(
            jax.ShapeDtypeStruct((EROWS, 128), _f32),
            jax.ShapeDtypeStruct((NC * NP,), _f32),
        ),
        mesh=_sc_mesh(),
        compiler_params=pltpu.CompilerParams(needs_layout_passes=False),
        scratch_types=(
            pltpu.VMEM((NP,), _f32),
            pltpu.VMEM((NP,), _f32),
            pltpu.VMEM((CHUNK_ROWS, 128), _i32),
            pltpu.VMEM((CHUNK_ROWS, 128), _i32),
            pltpu.VMEM((CHUNK_ROWS, 128), _f32),
            pltpu.VMEM((STRIPE,), _f32),
            pltpu.VMEM((16,), _f32),
            pltpu.VMEM_SHARED((NP,), _f32),
        ),
    )


def _edge_rows_body(g128, ee_hbm, src_hbm, dst_hbm, aout,
                    srcb, dstb, eeb, rows128, rows_v, zrows, zrows2, out_sh):
    c = lax.axis_index("c")
    t = lax.axis_index("s")
    row0 = t * (EROWS // NS)
    w = pl.multiple_of(c * HH, HH)

    if True:
        def zc(i, carry):
            zrows[i // 2, pl.ds((i % 2) * 16, 16)] = jnp.zeros((16,), _f32)
            return carry

        lax.fori_loop(0, 128, zc, 0)

        def zcopy(z, carry):
            pltpu.sync_copy(zrows, out_sh.at[pl.ds(t * STRIPE + z * 64, 64)])
            return carry

        lax.fori_loop(0, STRIPE // 64, zcopy, 0)
        plsc.subcore_barrier()

        def chunk(ci, carry):
            r = pl.multiple_of(row0 + ci * CHUNK_ROWS, 8)
            pltpu.sync_copy(src_hbm.at[pl.ds(r, CHUNK_ROWS)], srcb)
            pltpu.sync_copy(dst_hbm.at[pl.ds(r, CHUNK_ROWS)], dstb)
            pltpu.sync_copy(ee_hbm.at[pl.ds(r, CHUNK_ROWS)], eeb)

            def row128(j, carry2):
                pltpu.sync_copy(g128.at[srcb.at[j]], rows128)

                def grp16(g, carry3):
                    ev = eeb[j, pl.ds(g * 16, 16)]
                    for l in range(16):
                        e = g * 16 + l
                        es = jnp.full((16,), ev[l], _f32)
                        rows_v[e, pl.ds(0, 16)] = (
                            rows128[e, pl.ds(w, 16)] * es)
                        rows_v[e, pl.ds(16, 16)] = (
                            rows128[e, pl.ds(w + 16, 16)] * es)
                    return carry3

                lax.fori_loop(0, 8, grp16, 0)
                pltpu.sync_copy(rows_v, out_sh.at[dstb.at[j]], add=True)
                return carry2

            lax.fori_loop(0, CHUNK_ROWS, row128, 0)
            return carry

        lax.fori_loop(0, (EROWS // NS) // CHUNK_ROWS, chunk, 0)
        plsc.subcore_barrier()

        def ocopy(z, carry):
            off = t * STRIPE + z * 64
            pltpu.sync_copy(out_sh.at[pl.ds(off, 64)], zrows)

            def shuf(k2, carry2):
                f = k2 * 16
                zrows2[f // 128, pl.ds(f % 128, 16)] = (
                    zrows[f // 32, pl.ds(f % 32, 16)])
                return carry2

            lax.fori_loop(0, 128, shuf, 0)
            off4 = pl.multiple_of(c * (NP * HH // 128) + off // 4, 8)
            pltpu.sync_copy(zrows2, aout.at[pl.ds(off4, 16)])
            return carry

        lax.fori_loop(0, STRIPE // 64, ocopy, 0)


@functools.lru_cache(maxsize=None)
def _edge_rows_kernel():
    return pl.kernel(
        _edge_rows_body,
        out_type=jax.ShapeDtypeStruct((NC * NP * HH // 128, 128), _f32),
        mesh=_sc_mesh(),
        compiler_params=pltpu.CompilerParams(
            needs_layout_passes=False, use_tc_tiling_on_sc=False),
        scratch_types=(
            pltpu.VMEM((CHUNK_ROWS, 128), _i32),
            pltpu.VMEM((CHUNK_ROWS, 128), _i32),
            pltpu.VMEM((CHUNK_ROWS, 128), _f32),
            pltpu.VMEM((128, 128), _f32),
            pltpu.VMEM((128, HH), _f32),
            pltpu.VMEM((64, HH), _f32),
            pltpu.VMEM((16, 128), _f32),
            pltpu.VMEM_SHARED((NP, HH), _f32),
        ),
    )


def _lrelu(v):
    return jnp.where(v > 0, v, 0.2 * v)


def kernel(x, edge_index, batch, W1, b1, gamma, beta, Wc1, as1, ad1, bc1,
           Wc2, as2, ad2, bc2, Wg, bg, Wq1, bq1, Wq2, bq2):
    n = N_NODES
    loop = jnp.arange(n, dtype=edge_index.dtype)
    src = jnp.concatenate([edge_index[0], loop])
    dst = jnp.concatenate([edge_index[1], loop])
    pad_e = EP - src.shape[0]
    padv = jnp.full((pad_e,), n, _i32)
    src = jnp.concatenate([src, padv]).reshape(EROWS, 128)
    dst = jnp.concatenate([dst, padv]).reshape(EROWS, 128)

    x8 = jnp.zeros((NP, 8), _f32).at[:n, :5].set(x)
    W1p = jnp.zeros((8, H), _f32).at[:5].set(W1)

    def _layer(prep_out):
        g128, s, d, sm, dm = prep_out
        mvec = jnp.full((16,), _lrelu(sm[0, 0] + dm[0, 0]), _f32)
        ee, den = _edge_scalar_kernel()(
            s.reshape(NP), d.reshape(NP), src, dst, mvec)
        den = den.reshape(NC, NP)
        agg2 = _edge_rows_kernel()(g128, ee, src, dst).reshape(NC, NP, HH)
        return (agg2[0], agg2[1],
                den[0].reshape(NP, 1), den[1].reshape(NP, 1))

    prep1 = _prep_call(x8, W1p, b1.reshape(1, H), gamma.reshape(1, H),
                       beta.reshape(1, H), Wc1, as1.reshape(1, H),
                       ad1.reshape(1, H))
    alo1, ahi1, den10, den11 = _layer(prep1)

    prep2 = _mid_call(alo1, ahi1, den10, den11, bc1.reshape(1, H), Wc2,
                      as2.reshape(1, H), ad2.reshape(1, H))
    alo2, ahi2, den20, den21 = _layer(prep2)

    batchf = jnp.full((NP, 1), 1e9, _f32).at[:n, 0].set(batch.astype(_f32))
    h2, gate, gmax = _poolA_call(alo2, ahi2, den20, den21, bc2.reshape(1, H),
                                 Wg.reshape(1, H), bg.reshape(1, 1), batchf)
    q = _poolB_call(h2, gate, gmax, batchf, Wq1, bq1.reshape(1, H),
                    Wq2, bq2.reshape(1, A))
    return q
